# baseline (speedup 1.0000x reference)
import functools

import jax
import jax.numpy as jnp
from jax import lax
from jax.experimental import pallas as pl
from jax.experimental.pallas import tpu as pltpu
from jax.experimental.pallas import tpu_sc as plsc

_NC = 2
_NS = 16
_NW = _NC * _NS
_CHUNK = 128
_EPS = 1e-5
_TC_PARAMS = pltpu.CompilerParams(vmem_limit_bytes=100 * 1024 * 1024)

def _iota2(m, k, dim):
    return lax.broadcasted_iota(jnp.int32, (m, k), dim)


def _p_colfold():
    return (_iota2(128, 128, 0) % 8 == _iota2(128, 128, 1) % 8).astype(
        jnp.float32)


def _t_rep16():
    return (_iota2(8, 128, 0) == _iota2(8, 128, 1) % 8).astype(jnp.float32)


def _d_fold():
    return (_iota2(128, 8, 0) % 8 == _iota2(128, 8, 1)).astype(jnp.float32)


def _m_blkdiag():
    return (_iota2(128, 128, 0) // 8 == _iota2(128, 128, 1) // 8).astype(
        jnp.float32)


def _make_sc_scatter(n_pad, dout, tch):
    g_n = dout // 8
    nsub = _NW // g_n
    cps = tch // nsub
    k = 8
    ngrp = cps // k
    assert cps % k == 0
    mesh = plsc.VectorSubcoreMesh(core_axis_name="c", subcore_axis_name="s")

    @functools.partial(
        pl.kernel,
        out_type=jax.ShapeDtypeStruct((_NW, n_pad // 16, 128), jnp.float32),
        mesh=mesh,
        compiler_params=pltpu.CompilerParams(
            use_tc_tiling_on_sc=False, needs_layout_passes=False),
        scratch_types=[
            pltpu.VMEM((n_pad // 16, 128), jnp.float32),
            pltpu.VMEM((2, k, _CHUNK), jnp.int32),
            pltpu.VMEM((2, k, _CHUNK), jnp.int32),
            pltpu.VMEM((2 * k * _CHUNK, 8), jnp.float32),
            pltpu.SemaphoreType.DMA,
            pltpu.SemaphoreType.DMA,
        ],
    )
    def sc_kernel(hsf_hbm, src_hbm, dst_hbm, zeros_hbm, out_hbm, acc, idxs,
                  dsts, rows, sem_i, sem_g):
        c = lax.axis_index("c")
        s = lax.axis_index("s")
        w = c * _NS + s
        g = lax.rem(s, g_n)
        q = c * (_NS // g_n) + lax.div(s, g_n)
        base = q * cps
        goff = g * n_pad

        def issue_idx(t, p):
            pltpu.async_copy(src_hbm.at[pl.ds(base + t * k, k)], idxs.at[p],
                             sem_i)
            pltpu.async_copy(dst_hbm.at[pl.ds(base + t * k, k)], dsts.at[p],
                             sem_i)

        def wait_idx(p):
            pltpu.make_async_copy(src_hbm.at[pl.ds(0, k)], idxs.at[p],
                                  sem_i).wait()
            pltpu.make_async_copy(src_hbm.at[pl.ds(0, k)], dsts.at[p],
                                  sem_i).wait()

        def transform(p):
            if g_n > 1:
                for b in range(k):
                    for u in range(8):
                        sl = pl.ds(u * 16, 16)
                        idxs[p, b, sl] = idxs[p, b, sl] + goff

        def issue_g(t, p):
            for b in range(k):
                pltpu.async_copy(
                    hsf_hbm.at[idxs.at[p, b]],
                    rows.at[pl.ds((p * k + b) * _CHUNK, _CHUNK)], sem_g)

        iota16 = lax.iota(jnp.int32, 16)
        csplat = [jnp.full((16,), cc, jnp.int32) for cc in range(8)]

        def add_group(p):
            for b in range(k):
                rbase = (p * k + b) * _CHUNK
                for u in range(_CHUNK // 16):
                    d = dsts[p, b, pl.ds(u * 16, 16)]
                    d_hi = lax.shift_right_logical(d, 4)
                    d_lo = lax.shift_left(lax.bitwise_and(d, 15), 3)
                    ridx = iota16 + (rbase + u * 16)
                    for cc in range(8):
                        vals = plsc.load_gather(rows, [ridx, csplat[cc]])
                        plsc.addupdate_scatter(acc, [d_hi, d_lo + cc], vals)

        issue_idx(0, 0)
        wait_idx(0)
        transform(0)
        issue_g(0, 0)
        if ngrp > 1:
            issue_idx(1, 1)
        pltpu.sync_copy(zeros_hbm, acc)

        def steady(t, carry):
            p = lax.rem(t, 2)
            for b in range(k):
                pltpu.make_async_copy(
                    hsf_hbm.at[pl.ds(0, _CHUNK)],
                    rows.at[pl.ds((p * k + b) * _CHUNK, _CHUNK)],
                    sem_g).wait()

            @pl.when(t + 1 < ngrp)
            def _():
                wait_idx(1 - p)
                transform(1 - p)
                issue_g(t + 1, 1 - p)

            add_group(p)

            @pl.when(t + 2 < ngrp)
            def _():
                issue_idx(t + 2, p)

            return carry

        lax.fori_loop(0, ngrp, steady, 0)
        pltpu.sync_copy(acc, out_hbm.at[w])

    return sc_kernel


def _make_sc_degree(n_pad, tch):
    cpt = tch // _NW
    mesh = plsc.VectorSubcoreMesh(core_axis_name="c", subcore_axis_name="s")

    @functools.partial(
        pl.kernel,
        out_type=jax.ShapeDtypeStruct((_NW, n_pad // 16, 128), jnp.float32),
        mesh=mesh,
        compiler_params=pltpu.CompilerParams(
            use_tc_tiling_on_sc=False, needs_layout_passes=False),
        scratch_types=[
            pltpu.VMEM((n_pad // 16, 128), jnp.float32),
            pltpu.VMEM((cpt, _CHUNK), jnp.int32),
            pltpu.SemaphoreType.DMA,
        ],
    )
    def sc_kernel(dst_hbm, out_hbm, acc, dsts, sem_i):
        c = lax.axis_index("c")
        s = lax.axis_index("s")
        w = c * _NS + s
        cp = pltpu.async_copy(dst_hbm.at[pl.ds(w * cpt, cpt)], dsts, sem_i)

        def zero(j, carry):
            for u in range(8):
                acc[j, pl.ds(u * 16, 16)] = jnp.zeros((16,), jnp.float32)
            return carry

        lax.fori_loop(0, n_pad // 16, zero, 0)
        cp.wait()

        ones16 = jnp.ones((16,), jnp.float32)

        def body(j, carry):
            for u in range(8):
                d = dsts[j, pl.ds(u * 16, 16)]
                d_hi = lax.shift_right_logical(d, 4)
                d_lo = lax.shift_left(lax.bitwise_and(d, 15), 3)
                for cc in range(8):
                    plsc.addupdate_scatter(acc, [d_hi, d_lo + cc], ones16)
            return carry

        lax.fori_loop(0, cpt, body, 0)
        pltpu.sync_copy(acc, out_hbm.at[w])

    return sc_kernel


def _mask_pk(n, n_pad):
    r16 = lax.broadcasted_iota(jnp.int32, (n_pad // 16, 128), 0)
    sub = lax.broadcasted_iota(jnp.int32, (n_pad // 16, 128), 1) // 8
    return ((r16 * 16 + sub) < n).astype(jnp.float32)


def _grep(v, go):
    return jnp.dot(v[None, go * 8:go * 8 + 8], _t_rep16(),
                   preferred_element_type=jnp.float32)


def _bn_relu_pk(zs, n, g_vec, be_vec, mask):
    outs = []
    for go, z in enumerate(zs):
        zm = z * mask
        mu = jnp.dot(jnp.sum(zm, axis=0, keepdims=True), _p_colfold(),
                     preferred_element_type=jnp.float32) / n
        d = (z - mu) * mask
        var = jnp.dot(jnp.sum(d * d, axis=0, keepdims=True), _p_colfold(),
                      preferred_element_type=jnp.float32) / n
        y = (z - mu) * lax.rsqrt(var + _EPS) * _grep(g_vec, go) \
            + _grep(be_vec, go)
        outs.append(jnp.maximum(y, 0.0) * mask)
    return outs


def _mm_pk(us, w):
    g_in = len(us)
    g_out = w.shape[1] // 8
    outs = []
    for go in range(g_out):
        acc = None
        for g in range(g_in):
            wexp = jnp.dot(
                jnp.dot(_d_fold(), w[8 * g:8 * g + 8, 8 * go:8 * go + 8],
                        preferred_element_type=jnp.float32),
                _t_rep16(), preferred_element_type=jnp.float32) * _m_blkdiag()
            term = jnp.dot(us[g], wexp, preferred_element_type=jnp.float32)
            acc = term if acc is None else acc + term
        outs.append(acc)
    return outs


def _combine(acc_ref, hs_ref, n_pad, g_n):
    a = jnp.sum(acc_ref[...].reshape(_NC, _NS // g_n, g_n, n_pad // 16, 128),
                axis=(0, 1))
    r16 = n_pad // 16
    return [a[g] + hs_ref[pl.ds(g * r16, r16), :] for g in range(g_n)]


def _tc_mm1_body(x_ref, w_ref, h_ref):
    h_ref[...] = jnp.dot(x_ref[...], w_ref[...],
                         preferred_element_type=jnp.float32)


def _tc_scale_body(h_ref, deg_ref, dis_ref, hs_ref):
    deg = jnp.sum(deg_ref[...], axis=0) + 1.0
    dis = lax.rsqrt(deg)
    dis_ref[...] = dis
    hs_ref[...] = h_ref[...] * dis


def _tc_post1_body(n, n_pad, acc_ref, hs_ref, dis_ref, g_ref, be_ref,
                   out_ref):
    dis = dis_ref[...]
    zs = [ag * dis for ag in _combine(acc_ref, hs_ref, n_pad, 1)]
    y = _bn_relu_pk(zs, n, g_ref[...], be_ref[...], _mask_pk(n, n_pad))
    out_ref[...] = y[0] * dis


def _tc_mid_body(n, n_pad, g_n, acc_ref, hs_ref, dis_ref, g_ref, be_ref,
                 w_ref, out_ref):
    dis = dis_ref[...]
    zs = [ag * dis for ag in _combine(acc_ref, hs_ref, n_pad, g_n)]
    us = _mm_pk(zs, w_ref[...])
    ys = _bn_relu_pk(us, n, g_ref[...], be_ref[...], _mask_pk(n, n_pad))
    r16 = n_pad // 16
    for go, y in enumerate(ys):
        out_ref[pl.ds(go * r16, r16), :] = y * dis


def _tc_final_body(n, n_pad, g_n, acc_ref, hs_ref, dis_ref, g_ref, be_ref,
                   w_ref, wo_ref, bo_ref, out_ref):
    dis = dis_ref[...]
    zs = [ag * dis for ag in _combine(acc_ref, hs_ref, n_pad, g_n)]
    us = _mm_pk(zs, w_ref[...])
    ys = _bn_relu_pk(us, n, g_ref[...], be_ref[...], _mask_pk(n, n_pad))
    pooled = jnp.concatenate(
        [jnp.dot(jnp.sum(y, axis=0, keepdims=True), _d_fold(),
                 preferred_element_type=jnp.float32) for y in ys],
        axis=1) / n
    out_ref[...] = jnp.dot(
        pooled, wo_ref[...], preferred_element_type=jnp.float32) + bo_ref[...]


def _tc_mm1(n_pad, x_pad, w1):
    return pl.pallas_call(
        _tc_mm1_body,
        compiler_params=_TC_PARAMS,
        out_shape=jax.ShapeDtypeStruct((n_pad, w1.shape[1]), jnp.float32),
    )(x_pad, w1)


def _tc_scale(n_pad, h1_pk, deg_acc):
    r16 = n_pad // 16
    return pl.pallas_call(
        _tc_scale_body,
        compiler_params=_TC_PARAMS,
        out_shape=(
            jax.ShapeDtypeStruct((r16, 128), jnp.float32),
            jax.ShapeDtypeStruct((r16, 128), jnp.float32),
        ),
    )(h1_pk, deg_acc)


def _tc_post1(n, n_pad, acc, hs_pk, dis_pk, g, be):
    return pl.pallas_call(
        functools.partial(_tc_post1_body, n, n_pad),
        compiler_params=_TC_PARAMS,
        out_shape=jax.ShapeDtypeStruct((n_pad // 16, 128), jnp.float32),
    )(acc, hs_pk, dis_pk, g, be)


def _tc_mid(n, n_pad, acc, hs_pk, dis_pk, g, be, w):
    g_n = hs_pk.shape[0] // (n_pad // 16)
    g_out = w.shape[1] // 8
    return pl.pallas_call(
        functools.partial(_tc_mid_body, n, n_pad, g_n),
        compiler_params=_TC_PARAMS,
        out_shape=jax.ShapeDtypeStruct((g_out * (n_pad // 16), 128),
                                       jnp.float32),
    )(acc, hs_pk, dis_pk, g, be, w)


def _tc_final(n, n_pad, acc, hs_pk, dis_pk, g, be, w, wo, bo):
    g_n = hs_pk.shape[0] // (n_pad // 16)
    return pl.pallas_call(
        functools.partial(_tc_final_body, n, n_pad, g_n),
        compiler_params=_TC_PARAMS,
        out_shape=jax.ShapeDtypeStruct((1, wo.shape[1]), jnp.float32),
    )(acc, hs_pk, dis_pk, g, be, w, wo, bo)


def kernel(x, edge_index, W1, b1, g1, be1, W2, b2, g2, be2, W3, b3, g3, be3,
           W4, b4, g4, be4, Wo, bo):
    n, f = x.shape
    e = edge_index.shape[1]
    n_pad = ((n + 1 + 255) // 256) * 256
    r16 = n_pad // 16
    ep = -(-e // (_NW * _CHUNK * 8)) * (_NW * _CHUNK * 8)
    tch = ep // _CHUNK

    pad = jnp.full((ep - e,), n, dtype=jnp.int32)
    src = jnp.concatenate([edge_index[0], pad]).reshape(tch, _CHUNK)
    dst = jnp.concatenate([edge_index[1], pad]).reshape(tch, _CHUNK)
    x_pad = jnp.pad(x, ((0, n_pad - n), (0, 0)))
    zeros8 = jnp.zeros((r16, 128), jnp.float32)

    deg_acc = _make_sc_degree(n_pad, tch)(dst)
    h1 = _tc_mm1(n_pad, x_pad, W1)

    dis_pk, hs1_pk = _tc_scale(n_pad, h1.reshape(r16, 128), deg_acc)
    sc8 = _make_sc_scatter(n_pad, 8, tch)
    acc1 = sc8(hs1_pk.reshape(n_pad, 8), src, dst, zeros8)
    ys1_pk = _tc_post1(n, n_pad, acc1, hs1_pk, dis_pk, g1, be1)
    acc2 = sc8(ys1_pk.reshape(n_pad, 8), src, dst, zeros8)
    ys2_pk = _tc_mid(n, n_pad, acc2, ys1_pk, dis_pk, g2, be2, W2)
    acc3 = _make_sc_scatter(n_pad, 16, tch)(
        ys2_pk.reshape(2 * n_pad, 8), src, dst, zeros8)
    ys3_pk = _tc_mid(n, n_pad, acc3, ys2_pk, dis_pk, g3, be3, W3)
    acc4 = _make_sc_scatter(n_pad, 32, tch)(
        ys3_pk.reshape(4 * n_pad, 8), src, dst, zeros8)
    return _tc_final(n, n_pad, acc4, ys3_pk, dis_pk, g4, be4, W4, Wo, bo)


# speedup vs baseline: 1.4187x; 1.4187x over previous
import functools

import jax
import jax.numpy as jnp
from jax import lax
from jax.experimental import pallas as pl
from jax.experimental.pallas import tpu as pltpu
from jax.experimental.pallas import tpu_sc as plsc

_NC = 2
_NS = 16
_NW = _NC * _NS
_CHUNK = 128
_EPS = 1e-5


def _pick_group(ch, dout):
    budget = 300 * 1024
    for k in (10, 8, 5, 4, 2, 1):
        if ch % k == 0 and 2 * k * _CHUNK * dout * 4 <= budget:
            return k
    return 1


def _make_sc_scatter(n_pad, dout, ch):
    r = n_pad // _NS
    k = _pick_group(ch, dout)
    ng = ch // k
    mesh = plsc.VectorSubcoreMesh(core_axis_name="c", subcore_axis_name="s")

    @functools.partial(
        pl.kernel,
        out_type=jax.ShapeDtypeStruct((_NC * n_pad, dout), jnp.float32),
        mesh=mesh,
        compiler_params=pltpu.CompilerParams(use_tc_tiling_on_sc=False),
        scratch_types=[
            pltpu.VMEM_SHARED((n_pad, dout), jnp.float32),
            pltpu.VMEM((ch, _CHUNK), jnp.int32),
            pltpu.VMEM((ch, _CHUNK), jnp.int32),
            pltpu.VMEM((2, k, _CHUNK, dout), jnp.float32),
            pltpu.SemaphoreType.DMA,
            pltpu.SemaphoreType.DMA,
        ],
    )
    def sc_kernel(hs_hbm, src_hbm, dst_hbm, out_hbm, acc_sh, src_v, dst_v,
                  rows_v, sem_g, sem_s):
        c = lax.axis_index("c")
        s = lax.axis_index("s")
        w = c * _NS + s
        pltpu.sync_copy(hs_hbm.at[pl.ds(s * r, r)], acc_sh.at[pl.ds(s * r, r)])
        pltpu.sync_copy(src_hbm.at[w], src_v)
        pltpu.sync_copy(dst_hbm.at[w], dst_v)
        plsc.subcore_barrier()

        def issue(t, p):
            for b in range(k):
                pltpu.async_copy(hs_hbm.at[src_v.at[t * k + b]],
                                 rows_v.at[p, b], sem_g)

        def drain_g(p):
            for b in range(k):
                pltpu.make_async_copy(hs_hbm.at[pl.ds(0, _CHUNK)],
                                      rows_v.at[p, b], sem_g).wait()

        def scat(t, p):
            for b in range(k):
                pltpu.async_copy(rows_v.at[p, b],
                                 acc_sh.at[dst_v.at[t * k + b]], sem_s,
                                 add=True)

        def drain_s():
            for b in range(k):
                pltpu.make_async_copy(hs_hbm.at[pl.ds(0, _CHUNK)],
                                      rows_v.at[0, b], sem_s).wait()

        issue(0, 0)
        drain_g(0)
        scat(0, 0)
        if ng > 1:
            issue(1, 1)

            def steady(t, carry):
                p = lax.rem(t, 2)
                drain_g(p)
                scat(t, p)
                drain_s()
                issue(t + 1, 1 - p)
                return carry

            if ng > 2:
                lax.fori_loop(1, ng - 1, steady, 0)
            pl_ = (ng - 1) % 2
            drain_g(pl_)
            scat(ng - 1, pl_)
        for _ in range(min(ng, 2)):
            drain_s()
        plsc.subcore_barrier()
        pltpu.sync_copy(acc_sh.at[pl.ds(s * r, r)],
                        out_hbm.at[pl.ds(c * n_pad + s * r, r)])

    return sc_kernel


def _make_sc_degree(n_pad, dout, ch):
    r = n_pad // _NS
    mesh = plsc.VectorSubcoreMesh(core_axis_name="c", subcore_axis_name="s")

    @functools.partial(
        pl.kernel,
        out_type=jax.ShapeDtypeStruct((_NC * n_pad, dout), jnp.float32),
        mesh=mesh,
        compiler_params=pltpu.CompilerParams(use_tc_tiling_on_sc=False),
        scratch_types=[
            pltpu.VMEM_SHARED((n_pad, dout), jnp.float32),
            pltpu.VMEM((ch, _CHUNK), jnp.int32),
            pltpu.VMEM((_CHUNK, dout), jnp.float32),
            pltpu.SemaphoreType.DMA,
        ],
    )
    def sc_kernel(ones_hbm, dst_hbm, out_hbm, acc_sh, dst_v, rows_v, sem_s):
        c = lax.axis_index("c")
        s = lax.axis_index("s")
        w = c * _NS + s
        pltpu.sync_copy(ones_hbm.at[pl.ds(s * r, r)],
                        acc_sh.at[pl.ds(s * r, r)])
        pltpu.sync_copy(dst_hbm.at[w], dst_v)
        pltpu.sync_copy(ones_hbm.at[pl.ds(0, _CHUNK)], rows_v)
        plsc.subcore_barrier()

        def body(j, carry):
            pltpu.async_copy(rows_v, acc_sh.at[dst_v.at[j]], sem_s, add=True)
            return carry

        lax.fori_loop(0, ch, body, 0)

        def drain(j, carry):
            pltpu.make_async_copy(ones_hbm.at[pl.ds(0, _CHUNK)], rows_v,
                                  sem_s).wait()
            return carry

        lax.fori_loop(0, ch, drain, 0)
        plsc.subcore_barrier()
        pltpu.sync_copy(acc_sh.at[pl.ds(s * r, r)],
                        out_hbm.at[pl.ds(c * n_pad + s * r, r)])

    return sc_kernel


def _row_mask(n, n_pad):
    return (lax.broadcasted_iota(jnp.int32, (n_pad, 1), 0) < n).astype(
        jnp.float32)


def _bn_relu(z, n, n_pad, g, be):
    mask = _row_mask(n, n_pad)
    zm = z * mask
    mu = jnp.sum(zm, axis=0, keepdims=True) / n
    d = (z - mu) * mask
    var = jnp.sum(d * d, axis=0, keepdims=True) / n
    y = (z - mu) * lax.rsqrt(var + _EPS) * g + be
    return jnp.maximum(y, 0.0) * mask


def _tc_mm1_body(n, x_ref, w_ref, h_ref):
    h_ref[:n] = jnp.dot(x_ref[...], w_ref[...],
                        preferred_element_type=jnp.float32)
    h_ref[n:] = jnp.zeros((h_ref.shape[0] - n, h_ref.shape[1]), jnp.float32)


def _tc_scale_body(h_ref, deg_ref, dis_ref, hs_ref):
    n_pad = h_ref.shape[0]
    deg = deg_ref[:n_pad, 0:1] + deg_ref[n_pad:, 0:1] - 1.0
    dis = lax.rsqrt(deg)
    dis_ref[...] = dis
    hs_ref[...] = h_ref[...] * dis


def _tc_post1_body(n, acc_ref, hs_ref, dis_ref, g_ref, be_ref, out_ref):
    n_pad = hs_ref.shape[0]
    dis = dis_ref[...]
    agg = acc_ref[:n_pad] + acc_ref[n_pad:] - hs_ref[...]
    y = _bn_relu(agg * dis, n, n_pad, g_ref[...], be_ref[...])
    out_ref[...] = y * dis


def _tc_mid_body(n, acc_ref, hs_ref, dis_ref, g_ref, be_ref, w_ref, out_ref):
    n_pad = hs_ref.shape[0]
    dis = dis_ref[...]
    agg = acc_ref[:n_pad] + acc_ref[n_pad:] - hs_ref[...]
    u = jnp.dot(agg * dis, w_ref[...], preferred_element_type=jnp.float32)
    y = _bn_relu(u, n, n_pad, g_ref[...], be_ref[...])
    out_ref[...] = y * dis


def _tc_final_body(n, acc_ref, hs_ref, dis_ref, g_ref, be_ref, w_ref, wo_ref,
                   bo_ref, out_ref):
    n_pad = hs_ref.shape[0]
    agg = acc_ref[:n_pad] + acc_ref[n_pad:] - hs_ref[...]
    u = jnp.dot(agg * dis_ref[...], w_ref[...],
                preferred_element_type=jnp.float32)
    y = _bn_relu(u, n, n_pad, g_ref[...], be_ref[...])
    pooled = jnp.sum(y, axis=0, keepdims=True) / n
    out_ref[...] = jnp.dot(
        pooled, wo_ref[...], preferred_element_type=jnp.float32) + bo_ref[...]


def _tc_mm1(n, n_pad, x, w1):
    return pl.pallas_call(
        functools.partial(_tc_mm1_body, n),
        out_shape=jax.ShapeDtypeStruct((n_pad, w1.shape[1]), jnp.float32),
    )(x, w1)


def _tc_scale(n_pad, h1, deg_acc):
    return pl.pallas_call(
        _tc_scale_body,
        out_shape=(
            jax.ShapeDtypeStruct((n_pad, 1), jnp.float32),
            jax.ShapeDtypeStruct((n_pad, h1.shape[1]), jnp.float32),
        ),
    )(h1, deg_acc)


def _tc_post1(n, n_pad, acc, hs, dis, g, be):
    return pl.pallas_call(
        functools.partial(_tc_post1_body, n),
        out_shape=jax.ShapeDtypeStruct((n_pad, hs.shape[1]), jnp.float32),
    )(acc, hs, dis, g, be)


def _tc_mid(n, n_pad, acc, hs, dis, g, be, w):
    return pl.pallas_call(
        functools.partial(_tc_mid_body, n),
        out_shape=jax.ShapeDtypeStruct((n_pad, w.shape[1]), jnp.float32),
    )(acc, hs, dis, g, be, w)


def _tc_final(n, acc, hs, dis, g, be, w, wo, bo):
    return pl.pallas_call(
        functools.partial(_tc_final_body, n),
        out_shape=jax.ShapeDtypeStruct((1, wo.shape[1]), jnp.float32),
    )(acc, hs, dis, g, be, w, wo, bo)


def kernel(x, edge_index, W1, b1, g1, be1, W2, b2, g2, be2, W3, b3, g3, be3,
           W4, b4, g4, be4, Wo, bo):
    n, f = x.shape
    e = edge_index.shape[1]
    n_pad = ((n + 1 + 255) // 256) * 256
    ch = -(-e // (_NW * _CHUNK))
    ep = _NW * ch * _CHUNK

    pad = jnp.full((ep - e,), n, dtype=jnp.int32)
    src = jnp.concatenate([edge_index[0], pad]).reshape(_NW, ch, _CHUNK)
    dst = jnp.concatenate([edge_index[1], pad]).reshape(_NW, ch, _CHUNK)
    ones = jnp.ones((n_pad, 8), jnp.float32)
    deg_acc = _make_sc_degree(n_pad, 8, ch)(ones, dst)
    h1 = _tc_mm1(n, n_pad, x, W1)

    dis, hs1 = _tc_scale(n_pad, h1, deg_acc)
    sc8 = _make_sc_scatter(n_pad, 8, ch)
    acc1 = sc8(hs1, src, dst)
    ys1 = _tc_post1(n, n_pad, acc1, hs1, dis, g1, be1)
    acc2 = sc8(ys1, src, dst)
    ys2 = _tc_mid(n, n_pad, acc2, ys1, dis, g2, be2, W2)
    acc3 = _make_sc_scatter(n_pad, 16, ch)(ys2, src, dst)
    ys3 = _tc_mid(n, n_pad, acc3, ys2, dis, g3, be3, W3)
    acc4 = _make_sc_scatter(n_pad, 32, ch)(ys3, src, dst)
    return _tc_final(n, acc4, ys3, dis, g4, be4, W4, Wo, bo)
